# baseline (speedup 1.0000x reference)
import numpy as np
import jax
import jax.numpy as jnp
from jax import lax
from jax.experimental import pallas as pl
from jax.experimental.pallas import tpu as pltpu
from jax.experimental.pallas import tpu_sc as plsc

N_NODES = 10000
N_EDGES = 160000
K = 16
HID = 128
BLK = 1000

G = 4
D = HID // G
NG = K // G
NT = 16
EPT = 10240
E_PAD = NT * EPT
CH = 64
NCH = EPT // CH
NB = 2
RPT = 624
TAIL = N_NODES - NT * RPT
TRASH = N_NODES


def _sc_body(fh, srcs, dsts, tcb, zeros, out4,
             idx2, dstb0, dstb1, ihb0, ihb1, tbuf0, tbuf1, bas0, bas1,
             rows0, rows1, sc0, sc1,
             accA, accB, accC, accD,
             semt0, semt1, semg0, semg1, semsc0, semsc1, semd0, semd1):
    c = lax.axis_index("c")
    s = lax.axis_index("s")
    f32 = jnp.float32
    dstbs = (dstb0, dstb1)
    ihbs = (ihb0, ihb1)
    tbufs = (tbuf0, tbuf1)
    bass = (bas0, bas1)
    rowss = (rows0, rows1)
    scs = (sc0, sc1)
    accs = (accA, accB, accC, accD)
    semts = (semt0, semt1)
    semgs = (semg0, semg1)
    semscs = (semsc0, semsc1)
    semds = (semd0, semd1)

    pltpu.sync_copy(srcs.at[c * NT + s], idx2)

    def _zero_acc():
        for kk in range(G):
            pltpu.sync_copy(zeros.at[pl.ds(0, RPT), :],
                            accs[kk].at[pl.ds(s * RPT, RPT), :])

            @pl.when(s == 0)
            def _():
                pltpu.sync_copy(zeros.at[pl.ds(0, TAIL + 8), :],
                                accs[kk].at[pl.ds(NT * RPT, TAIL + 8), :])

    def _basis(kg, tb, ba):
        for g in range(CH // 16):
            sl = pl.ds(g * 16, 16)
            t0 = tb[0, sl]
            t1 = tb[1, sl]
            com = jnp.ones((16,), f32)
            for d in (2, 3):
                bf = ((kg >> (d - 2)) & 1).astype(f32)
                td = tb[d, sl]
                com = com * ((1.0 - td) + bf * (2.0 * td - 1.0))
            f0 = (1.0 - t0, t0)
            f1 = (1.0 - t1, t1)
            for kk in range(G):
                ba[kk, sl] = com * f0[kk & 1] * f1[kk >> 1]

    def _scale(ba, rw, sc):
        @plsc.parallel_loop(0, CH, step=1, unroll=4)
        def scale(e):
            vs = [rw[e, pl.ds(u * 16, 16)] for u in range(D // 16)]
            for kk in range(G):
                bv = ba[kk, pl.ds(e, 16)][0]
                for u in range(D // 16):
                    sc[kk, e, pl.ds(u * 16, 16)] = vs[u] * bv

    def _drain_scatter(b):
        for kk in range(G):
            pltpu.make_async_copy(zeros.at[pl.ds(0, CH), :],
                                  scs[b].at[kk], semscs[b]).wait()

    def one_pass(p, _):
        kg = p >> 2
        q = p & 3
        _zero_acc()
        plsc.subcore_barrier()

        def batch(bt, _):
            dts, dgs, dds = [], [], []
            for b in range(NB):
                j = bt * NB + b

                @pl.when(bt > 0)
                def _():
                    _drain_scatter(b)
                dts.append(pltpu.async_copy(tcb.at[s, j], tbufs[b], semts[b]))
                dds.append(pltpu.async_copy(
                    dsts.at[pl.ds(s * EPT + j * CH, CH)], dstbs[b], semds[b]))
                for v in range(CH // 16):
                    sl = pl.ds(v * 16, 16)
                    ihbs[b][sl] = idx2[j, sl] + q
                dgs.append(pltpu.async_copy(fh.at[ihbs[b]], rowss[b],
                                            semgs[b]))
            for b in range(NB):
                dts[b].wait()
                _basis(kg, tbufs[b], bass[b])
                dgs[b].wait()
                _scale(bass[b], rowss[b], scs[b])
                dds[b].wait()
                for kk in range(G):
                    pltpu.async_copy(scs[b].at[kk], accs[kk].at[dstbs[b]],
                                     semscs[b], add=True)
            return 0
        lax.fori_loop(0, NCH // NB, batch, 0)
        for b in range(NB):
            _drain_scatter(b)

        plsc.subcore_barrier()
        for kk in range(G):
            task = c * K + kg * G + kk
            pltpu.sync_copy(accs[kk].at[pl.ds(s * RPT, RPT), :],
                            out4.at[task, q, pl.ds(s * RPT, RPT), :])

            @pl.when(s == 0)
            def _():
                pltpu.sync_copy(accs[kk].at[pl.ds(NT * RPT, TAIL), :],
                                out4.at[task, q, pl.ds(NT * RPT, TAIL), :])
        return 0

    lax.fori_loop(0, K, one_pass, 0)

    _zero_acc()
    plsc.subcore_barrier()

    def _fill_ones(r, _):
        for u in range(D // 16):
            sc0[0, r, pl.ds(u * 16, 16)] = jnp.ones((16,), f32)
        return 0
    lax.fori_loop(0, CH, _fill_ones, 0)

    def dchunk(j, _):
        pltpu.async_copy(dsts.at[pl.ds(s * EPT + j * CH, CH)], dstb0,
                         semd0).wait()
        pltpu.sync_copy(sc0.at[0], accA.at[dstb0], add=True)
        return 0
    lax.fori_loop(0, NCH, dchunk, 0)
    plsc.subcore_barrier()

    @pl.when(c == 0)
    def _():
        pltpu.sync_copy(accA.at[pl.ds(s * RPT, RPT), :],
                        out4.at[2 * K, 0, pl.ds(s * RPT, RPT), :])

    @pl.when(jnp.logical_and(c == 0, s == 0))
    def _():
        pltpu.sync_copy(accA.at[pl.ds(NT * RPT, TAIL), :],
                        out4.at[2 * K, 0, pl.ds(NT * RPT, TAIL), :])


_sc_scatter = pl.kernel(
    _sc_body,
    mesh=plsc.VectorSubcoreMesh(core_axis_name="c", subcore_axis_name="s"),
    compiler_params=pltpu.CompilerParams(use_tc_tiling_on_sc=False),
    out_type=jax.ShapeDtypeStruct((2 * K + 1, G, N_NODES, D), jnp.float32),
    scratch_types=[
        pltpu.VMEM((NCH, CH), jnp.int32),
        pltpu.VMEM((CH,), jnp.int32),
        pltpu.VMEM((CH,), jnp.int32),
        pltpu.VMEM((CH,), jnp.int32),
        pltpu.VMEM((CH,), jnp.int32),
        pltpu.VMEM((4, CH), jnp.float32),
        pltpu.VMEM((4, CH), jnp.float32),
        pltpu.VMEM((G, CH + 16), jnp.float32),
        pltpu.VMEM((G, CH + 16), jnp.float32),
        pltpu.VMEM((CH, D), jnp.float32),
        pltpu.VMEM((CH, D), jnp.float32),
        pltpu.VMEM((G, CH, D), jnp.float32),
        pltpu.VMEM((G, CH, D), jnp.float32),
        pltpu.VMEM_SHARED((N_NODES + 8, D), jnp.float32),
        pltpu.VMEM_SHARED((N_NODES + 8, D), jnp.float32),
        pltpu.VMEM_SHARED((N_NODES + 8, D), jnp.float32),
        pltpu.VMEM_SHARED((N_NODES + 8, D), jnp.float32),
        pltpu.SemaphoreType.DMA,
        pltpu.SemaphoreType.DMA,
        pltpu.SemaphoreType.DMA,
        pltpu.SemaphoreType.DMA,
        pltpu.SemaphoreType.DMA,
        pltpu.SemaphoreType.DMA,
        pltpu.SemaphoreType.DMA,
        pltpu.SemaphoreType.DMA,
    ],
)


def _dense_body(accx_ref, acch_ref, deg_ref, x_ref, h_ref,
                wx_ref, wh_ref, rx_ref, rh_ref, bx_ref, bh_ref, out_ref):
    f32 = jnp.float32
    dn = (((1,), (0,)), ((), ()))
    ax = lax.dot_general(x_ref[...], rx_ref[...], dn, preferred_element_type=f32)
    ah = lax.dot_general(h_ref[...], rh_ref[...], dn, preferred_element_type=f32)
    axs = jnp.zeros_like(ax)
    ahs = jnp.zeros_like(ah)
    for k in range(K):
        axs += lax.dot_general(accx_ref[k], wx_ref[k], dn, preferred_element_type=f32)
        ahs += lax.dot_general(acch_ref[k], wh_ref[k], dn, preferred_element_type=f32)
    dinv = 1.0 / jnp.maximum(deg_ref[0][:, 0:1], 1.0)
    ax = ax + axs * dinv + bx_ref[...]
    ah = ah + ahs * dinv + bh_ref[...]
    xr_o = ax[:, 0:128]
    xz_o = ax[:, 128:256]
    xn_o = ax[:, 256:384]
    hr_o = ah[:, 0:128]
    hz_o = ah[:, 128:256]
    r = jax.nn.sigmoid(xr_o + hr_o)
    z = jax.nn.sigmoid(xz_o + hz_o)
    n = jnp.tanh(xn_o + r * hr_o)
    out_ref[...] = (1.0 - z) * n + z * h_ref[...]


def _dense_phase(accs, x, hidden, wx, wh, rx, rh, bx, bh):
    grid = (N_NODES // BLK,)
    return pl.pallas_call(
        _dense_body,
        grid=grid,
        in_specs=[
            pl.BlockSpec((K, BLK, HID), lambda i: (0, i, 0)),
            pl.BlockSpec((K, BLK, HID), lambda i: (1, i, 0)),
            pl.BlockSpec((1, BLK, HID), lambda i: (2 * K, i, 0)),
            pl.BlockSpec((BLK, HID), lambda i: (i, 0)),
            pl.BlockSpec((BLK, HID), lambda i: (i, 0)),
            pl.BlockSpec((K, HID, 3 * HID), lambda i: (0, 0, 0)),
            pl.BlockSpec((K, HID, 2 * HID), lambda i: (0, 0, 0)),
            pl.BlockSpec((HID, 3 * HID), lambda i: (0, 0)),
            pl.BlockSpec((HID, 2 * HID), lambda i: (0, 0)),
            pl.BlockSpec((1, 3 * HID), lambda i: (0, 0)),
            pl.BlockSpec((1, 2 * HID), lambda i: (0, 0)),
        ],
        out_specs=pl.BlockSpec((BLK, HID), lambda i: (i, 0)),
        out_shape=jax.ShapeDtypeStruct((N_NODES, HID), jnp.float32),
    )(accs, accs, accs, x, hidden, wx, wh, rx, rh, bx, bh)


def kernel(x, hidden, edge_index, edge_attr,
           xr_w, xr_root, xr_b, hr_w, hr_root, hr_b,
           xz_w, xz_root, xz_b, hz_w, hz_root, hz_b,
           xn_w, xn_root, xn_b):
    src = edge_index[0].astype(jnp.int32)
    dst = edge_index[1].astype(jnp.int32)
    npad = E_PAD - N_EDGES
    srcp = jnp.concatenate([src, jnp.zeros((npad,), jnp.int32)])
    dstp = jnp.concatenate([dst, jnp.full((npad,), TRASH, jnp.int32)])
    eatp = jnp.concatenate([edge_attr,
                            jnp.full((npad, 4), 0.5, jnp.float32)])
    feats = jnp.concatenate([x, hidden], axis=0)
    fh = feats.reshape(2 * N_NODES * G, D)
    srcs = (jnp.stack([srcp, srcp + N_NODES]) * G).reshape(2 * NT, NCH, CH)
    dsts = dstp
    tcb = eatp.reshape(NT, NCH, CH, 4).transpose(0, 1, 3, 2)
    zeros = jnp.zeros((RPT + TAIL + 8, D), jnp.float32)

    out4 = _sc_scatter(fh, srcs, dsts, tcb, zeros)
    accs = out4.transpose(0, 2, 1, 3).reshape(2 * K + 1, N_NODES, HID)

    wx = jnp.concatenate([xr_w, xz_w, xn_w], axis=2)
    wh = jnp.concatenate([hr_w, hz_w], axis=2)
    rx = jnp.concatenate([xr_root, xz_root, xn_root], axis=1)
    rh = jnp.concatenate([hr_root, hz_root], axis=1)
    bx = jnp.concatenate([xr_b, xz_b, xn_b])[None, :]
    bh = jnp.concatenate([hr_b, hz_b])[None, :]
    return _dense_phase(accs, x, hidden, wx, wh, rx, rh, bx, bh)


# speedup vs baseline: 1.1137x; 1.1137x over previous
import numpy as np
import jax
import jax.numpy as jnp
from jax import lax
from jax.experimental import pallas as pl
from jax.experimental.pallas import tpu as pltpu
from jax.experimental.pallas import tpu_sc as plsc

N_NODES = 10000
N_EDGES = 160000
K = 16
HID = 128
HIDP = HID // 2
BLK = 1000

NT = 16
EPT = N_EDGES // NT
CH = 80
NCH = EPT // CH
NB = 2
RPT = 624
TAIL = N_NODES - NT * RPT

_PERM = np.arange(HID).reshape(4, 2, 16).transpose(0, 2, 1).reshape(HID)


def _sc_body(feats, srcs, dsts, tcb, zeros, accs_out,
             srcb0, srcb1, dstb0, dstb1, tbuf0, tbuf1, bas0, bas1,
             rows0, rows1, stg0, stg1, acc_sh,
             semt0, semt1, semg0, semg1, semsc0, semsc1,
             semd0, semd1, sems0, sems1):
    c = lax.axis_index("c")
    s = lax.axis_index("s")
    f32 = jnp.float32
    srcbs = (srcb0, srcb1)
    dstbs = (dstb0, dstb1)
    tbufs = (tbuf0, tbuf1)
    bass = (bas0, bas1)
    rowss = (rows0, rows1)
    stgs = (stg0, stg1)
    semts = (semt0, semt1)
    semgs = (semg0, semg1)
    semscs = (semsc0, semsc1)
    semds = (semd0, semd1)
    semss = (sems0, sems1)

    def _zero_acc():
        pltpu.sync_copy(zeros.at[pl.ds(0, RPT), :],
                        acc_sh.at[pl.ds(s * RPT, RPT), :])

        @pl.when(s == 0)
        def _():
            pltpu.sync_copy(zeros.at[pl.ds(0, TAIL), :],
                            acc_sh.at[pl.ds(NT * RPT, TAIL), :])

    def _writeback(task):
        pltpu.sync_copy(acc_sh.at[pl.ds(s * RPT, RPT), :],
                        accs_out.at[task, pl.ds(s * RPT, RPT), :])

        @pl.when(s == 0)
        def _():
            pltpu.sync_copy(acc_sh.at[pl.ds(NT * RPT, TAIL), :],
                            accs_out.at[task, pl.ds(NT * RPT, TAIL), :])

    def _basis(p, tb, ba):
        for g in range(CH // 16):
            b = jnp.ones((16,), f32)
            for d in range(4):
                bf = ((p >> d) & 1).astype(f32)
                td = tb[d, pl.ds(g * 16, 16)]
                b = b * ((1.0 - td) + bf * (2.0 * td - 1.0))
            ba[pl.ds(g * 16, 16)] = b

    def _scale(ba, rw, stg):
        @plsc.parallel_loop(0, CH, step=1, unroll=4)
        def scale(e):
            bv = ba[pl.ds(e, 16)][0]
            for u in range(4):
                v32 = rw[e, pl.ds(u * 16, 16)]
                a = lax.bitcast_convert_type(v32 << 16, f32)
                b = lax.bitcast_convert_type(v32 & jnp.int32(-65536), f32)
                stg[e, pl.ds(u * 32, 16)] = a * bv
                stg[e, pl.ds(u * 32 + 16, 16)] = b * bv

    def _drain_scatter(b):
        pltpu.make_async_copy(zeros.at[pl.ds(0, CH), :], stgs[b],
                              semscs[b]).wait()

    def _issue_loads(b, j):
        dt = pltpu.async_copy(tcb.at[s, j], tbufs[b], semts[b])
        dd = pltpu.async_copy(dsts.at[pl.ds(s * EPT + j * CH, CH)],
                              dstbs[b], semds[b])
        dsr = pltpu.async_copy(srcs.at[pl.ds(c * N_EDGES + s * EPT + j * CH,
                                             CH)], srcbs[b], semss[b])
        return dt, dd, dsr

    def one_pass(p, _):
        _zero_acc()
        plsc.subcore_barrier()

        def batch(q, _):
            descs = []
            for b in range(NB):
                j = q * NB + b

                @pl.when(q > 0)
                def _():
                    _drain_scatter(b)
                descs.append(_issue_loads(b, j))
            for b in range(NB):
                dt, dd, dsr = descs[b]
                dsr.wait()
                dg = pltpu.async_copy(feats.at[srcbs[b]], rowss[b], semgs[b])
                dt.wait()
                _basis(p, tbufs[b], bass[b])
                dg.wait()
                _scale(bass[b], rowss[b], stgs[b])
                dd.wait()
                pltpu.async_copy(stgs[b], acc_sh.at[dstbs[b]], semscs[b],
                                 add=True)
            return 0
        lax.fori_loop(0, NCH // NB, batch, 0)
        for b in range(NB):
            _drain_scatter(b)

        jt = (NCH // NB) * NB
        dt, dd, dsr = _issue_loads(0, jt)
        dsr.wait()
        dg = pltpu.async_copy(feats.at[srcb0], rows0, semg0)
        dt.wait()
        _basis(p, tbuf0, bas0)
        dg.wait()
        _scale(bas0, rows0, stg0)
        dd.wait()
        pltpu.async_copy(stg0, acc_sh.at[dstb0], semsc0, add=True)
        _drain_scatter(0)

        plsc.subcore_barrier()
        _writeback(c * K + p)
        return 0

    lax.fori_loop(0, K, one_pass, 0)

    _zero_acc()
    plsc.subcore_barrier()

    def _fill_ones(r, _):
        for u in range(8):
            stg0[r, pl.ds(u * 16, 16)] = jnp.ones((16,), f32)
        return 0
    lax.fori_loop(0, CH, _fill_ones, 0)

    def dchunk(j, _):
        pltpu.async_copy(dsts.at[pl.ds(s * EPT + j * CH, CH)], dstb0,
                         semd0).wait()
        pltpu.sync_copy(stg0, acc_sh.at[dstb0], add=True)
        return 0
    lax.fori_loop(0, NCH, dchunk, 0)
    plsc.subcore_barrier()

    @pl.when(c == 0)
    def _():
        _writeback(2 * K)


_sc_scatter = pl.kernel(
    _sc_body,
    mesh=plsc.VectorSubcoreMesh(core_axis_name="c", subcore_axis_name="s"),
    compiler_params=pltpu.CompilerParams(use_tc_tiling_on_sc=False),
    out_type=jax.ShapeDtypeStruct((2 * K + 1, N_NODES, HID), jnp.float32),
    scratch_types=[
        pltpu.VMEM((CH,), jnp.int32),
        pltpu.VMEM((CH,), jnp.int32),
        pltpu.VMEM((CH,), jnp.int32),
        pltpu.VMEM((CH,), jnp.int32),
        pltpu.VMEM((4, CH), jnp.float32),
        pltpu.VMEM((4, CH), jnp.float32),
        pltpu.VMEM((CH + 16,), jnp.float32),
        pltpu.VMEM((CH + 16,), jnp.float32),
        pltpu.VMEM((CH, HIDP), jnp.int32),
        pltpu.VMEM((CH, HIDP), jnp.int32),
        pltpu.VMEM((CH, HID), jnp.float32),
        pltpu.VMEM((CH, HID), jnp.float32),
        pltpu.VMEM_SHARED((N_NODES, HID), jnp.float32),
        pltpu.SemaphoreType.DMA,
        pltpu.SemaphoreType.DMA,
        pltpu.SemaphoreType.DMA,
        pltpu.SemaphoreType.DMA,
        pltpu.SemaphoreType.DMA,
        pltpu.SemaphoreType.DMA,
        pltpu.SemaphoreType.DMA,
        pltpu.SemaphoreType.DMA,
        pltpu.SemaphoreType.DMA,
        pltpu.SemaphoreType.DMA,
    ],
)


def _dense_body(accx_ref, acch_ref, deg_ref, x_ref, h_ref,
                wx_ref, wh_ref, rx_ref, rh_ref, bx_ref, bh_ref, out_ref):
    f32 = jnp.float32
    dn = (((1,), (0,)), ((), ()))
    ax = lax.dot_general(x_ref[...], rx_ref[...], dn, preferred_element_type=f32)
    ah = lax.dot_general(h_ref[...], rh_ref[...], dn, preferred_element_type=f32)
    axs = jnp.zeros_like(ax)
    ahs = jnp.zeros_like(ah)
    for k in range(K):
        axs += lax.dot_general(accx_ref[k], wx_ref[k], dn, preferred_element_type=f32)
        ahs += lax.dot_general(acch_ref[k], wh_ref[k], dn, preferred_element_type=f32)
    dinv = 1.0 / jnp.maximum(deg_ref[0][:, 0:1], 1.0)
    ax = ax + axs * dinv + bx_ref[...]
    ah = ah + ahs * dinv + bh_ref[...]
    xr_o = ax[:, 0:128]
    xz_o = ax[:, 128:256]
    xn_o = ax[:, 256:384]
    hr_o = ah[:, 0:128]
    hz_o = ah[:, 128:256]
    r = jax.nn.sigmoid(xr_o + hr_o)
    z = jax.nn.sigmoid(xz_o + hz_o)
    n = jnp.tanh(xn_o + r * hr_o)
    out_ref[...] = (1.0 - z) * n + z * h_ref[...]


def _dense_phase(accs, x, hidden, wx, wh, rx, rh, bx, bh):
    grid = (N_NODES // BLK,)
    return pl.pallas_call(
        _dense_body,
        grid=grid,
        in_specs=[
            pl.BlockSpec((K, BLK, HID), lambda i: (0, i, 0)),
            pl.BlockSpec((K, BLK, HID), lambda i: (1, i, 0)),
            pl.BlockSpec((1, BLK, HID), lambda i: (2 * K, i, 0)),
            pl.BlockSpec((BLK, HID), lambda i: (i, 0)),
            pl.BlockSpec((BLK, HID), lambda i: (i, 0)),
            pl.BlockSpec((K, HID, 3 * HID), lambda i: (0, 0, 0)),
            pl.BlockSpec((K, HID, 2 * HID), lambda i: (0, 0, 0)),
            pl.BlockSpec((HID, 3 * HID), lambda i: (0, 0)),
            pl.BlockSpec((HID, 2 * HID), lambda i: (0, 0)),
            pl.BlockSpec((1, 3 * HID), lambda i: (0, 0)),
            pl.BlockSpec((1, 2 * HID), lambda i: (0, 0)),
        ],
        out_specs=pl.BlockSpec((BLK, HID), lambda i: (i, 0)),
        out_shape=jax.ShapeDtypeStruct((N_NODES, HID), jnp.float32),
    )(accs, accs, accs, x, hidden, wx, wh, rx, rh, bx, bh)


def kernel(x, hidden, edge_index, edge_attr,
           xr_w, xr_root, xr_b, hr_w, hr_root, hr_b,
           xz_w, xz_root, xz_b, hz_w, hz_root, hz_b,
           xn_w, xn_root, xn_b):
    src = edge_index[0].astype(jnp.int32)
    dst = edge_index[1].astype(jnp.int32)
    feats = jnp.concatenate([x, hidden], axis=0)
    fperm = feats[:, jnp.asarray(_PERM)].astype(jnp.bfloat16)
    fpack = lax.bitcast_convert_type(
        fperm.reshape(2 * N_NODES, HIDP, 2), jnp.int32)
    srcs = jnp.concatenate([src, src + N_NODES])
    dsts = dst
    tcb = edge_attr.reshape(NT, NCH, CH, 4).transpose(0, 1, 3, 2)
    zeros = jnp.zeros((RPT + TAIL, HID), jnp.float32)

    accs = _sc_scatter(fpack, srcs, dsts, tcb, zeros)

    wx = jnp.concatenate([xr_w, xz_w, xn_w], axis=2)
    wh = jnp.concatenate([hr_w, hz_w], axis=2)
    rx = jnp.concatenate([xr_root, xz_root, xn_root], axis=1)
    rh = jnp.concatenate([hr_root, hz_root], axis=1)
    bx = jnp.concatenate([xr_b, xz_b, xn_b])[None, :]
    bh = jnp.concatenate([hr_b, hz_b])[None, :]
    return _dense_phase(accs, x, hidden, wx, wh, rx, rh, bx, bh)
